# baseline (speedup 1.0000x reference)
import functools

import jax
import jax.numpy as jnp
from jax.experimental import pallas as pl
from jax.experimental.pallas import tpu as pltpu

N_EXPERTS = 16
LBL_COEF = 0.01

BLOCK_T = 1024


def _router_half(logits, probs_ref, idx_ref):
    row = jax.lax.broadcasted_iota(jnp.int32, logits.shape, 0)
    big = jnp.int32(N_EXPERTS)

    m1 = jnp.max(logits, axis=0, keepdims=True)
    i1 = jnp.min(jnp.where(logits == m1, row, big), axis=0, keepdims=True)
    masked = jnp.where(row == i1, -jnp.inf, logits)
    m2 = jnp.max(masked, axis=0, keepdims=True)
    i2 = jnp.min(jnp.where(masked == m2, row, big), axis=0, keepdims=True)

    e2 = jnp.exp(m2 - m1)
    denom = 1.0 + e2
    probs_ref[...] = jnp.concatenate([1.0 / denom, e2 / denom], axis=0)
    idx_ref[...] = jnp.concatenate([i1, i2], axis=0)

    ex = jnp.exp(logits - m1)
    rp = ex / jnp.sum(ex, axis=0, keepdims=True)
    return jnp.sum(rp, axis=1, keepdims=True)


def _router_kernel(xa_ref, xb_ref, w_ref, pa_ref, ia_ref, pb_ref, ib_ref,
                   loss_ref, acc_ref, *, n_steps, n_tokens):
    step = pl.program_id(0)

    @pl.when(step == 0)
    def _init():
        acc_ref[...] = jnp.zeros_like(acc_ref)

    w = w_ref[...]
    la = jax.lax.dot_general(
        w, xa_ref[0],
        dimension_numbers=(((1,), (1,)), ((), ())),
        preferred_element_type=jnp.float32,
    )
    lb = jax.lax.dot_general(
        w, xb_ref[0],
        dimension_numbers=(((1,), (1,)), ((), ())),
        preferred_element_type=jnp.float32,
    )

    sa = _router_half(la, pa_ref, ia_ref)
    sb = _router_half(lb, pb_ref, ib_ref)
    acc_ref[...] += sa + sb

    @pl.when(step == n_steps - 1)
    def _finish():
        ep = acc_ref[...] / jnp.float32(n_tokens)
        loss_ref[0, 0] = LBL_COEF * jnp.sum(ep * jnp.log(ep + 1e-8))


def kernel(x, W):
    b, s, d = x.shape
    n_tokens = b * s
    half = n_tokens // 2
    xr = x.reshape(2, half, d)
    n_steps = half // BLOCK_T

    pa, ia, pb, ib, loss = pl.pallas_call(
        functools.partial(_router_kernel, n_steps=n_steps, n_tokens=n_tokens),
        grid=(n_steps,),
        in_specs=[
            pl.BlockSpec((1, BLOCK_T, d), lambda i: (0, i, 0)),
            pl.BlockSpec((1, BLOCK_T, d), lambda i: (1, i, 0)),
            pl.BlockSpec((N_EXPERTS, d), lambda i: (0, 0)),
        ],
        out_specs=[
            pl.BlockSpec((2, BLOCK_T), lambda i: (0, i)),
            pl.BlockSpec((2, BLOCK_T), lambda i: (0, i)),
            pl.BlockSpec((2, BLOCK_T), lambda i: (0, i)),
            pl.BlockSpec((2, BLOCK_T), lambda i: (0, i)),
            pl.BlockSpec(memory_space=pltpu.SMEM),
        ],
        out_shape=[
            jax.ShapeDtypeStruct((2, half), jnp.float32),
            jax.ShapeDtypeStruct((2, half), jnp.int32),
            jax.ShapeDtypeStruct((2, half), jnp.float32),
            jax.ShapeDtypeStruct((2, half), jnp.int32),
            jax.ShapeDtypeStruct((1, 1), jnp.float32),
        ],
        scratch_shapes=[pltpu.VMEM((N_EXPERTS, 1), jnp.float32)],
    )(xr, xr, W)

    probs = jnp.concatenate([pa, pb], axis=1)
    idx = jnp.concatenate([ia, ib], axis=1)
    return (probs.T.reshape(b, s, 2), idx.T.reshape(b, s, 2), loss[0, 0])


# speedup vs baseline: 1.0403x; 1.0403x over previous
import functools

import jax
import jax.numpy as jnp
from jax.experimental import pallas as pl
from jax.experimental.pallas import tpu as pltpu

N_EXPERTS = 16
LBL_COEF = 0.01

BLOCK_T = 512


def _router_half(logits, probs_ref, idx_ref):
    row = jax.lax.broadcasted_iota(jnp.int32, logits.shape, 0)
    big = jnp.int32(N_EXPERTS)

    m1 = jnp.max(logits, axis=0, keepdims=True)
    i1 = jnp.min(jnp.where(logits == m1, row, big), axis=0, keepdims=True)
    masked = jnp.where(row == i1, -jnp.inf, logits)
    m2 = jnp.max(masked, axis=0, keepdims=True)
    i2 = jnp.min(jnp.where(masked == m2, row, big), axis=0, keepdims=True)

    e2 = jnp.exp(m2 - m1)
    denom = 1.0 + e2
    probs_ref[...] = jnp.concatenate([1.0 / denom, e2 / denom], axis=0)
    idx_ref[...] = jnp.concatenate([i1, i2], axis=0)

    ex = jnp.exp(logits - m1)
    rp = ex / jnp.sum(ex, axis=0, keepdims=True)
    return jnp.sum(rp, axis=1, keepdims=True)


def _router_kernel(xa_ref, xb_ref, w_ref, pa_ref, ia_ref, pb_ref, ib_ref,
                   loss_ref, acc_ref, *, n_steps, n_tokens):
    step = pl.program_id(0)

    @pl.when(step == 0)
    def _init():
        acc_ref[...] = jnp.zeros_like(acc_ref)

    w = w_ref[...]
    la = jax.lax.dot_general(
        w, xa_ref[0],
        dimension_numbers=(((1,), (1,)), ((), ())),
        preferred_element_type=jnp.float32,
    )
    lb = jax.lax.dot_general(
        w, xb_ref[0],
        dimension_numbers=(((1,), (1,)), ((), ())),
        preferred_element_type=jnp.float32,
    )

    sa = _router_half(la, pa_ref, ia_ref)
    sb = _router_half(lb, pb_ref, ib_ref)
    acc_ref[...] += sa + sb

    @pl.when(step == n_steps - 1)
    def _finish():
        ep = acc_ref[...] / jnp.float32(n_tokens)
        loss_ref[0, 0] = LBL_COEF * jnp.sum(ep * jnp.log(ep + 1e-8))


def kernel(x, W):
    b, s, d = x.shape
    n_tokens = b * s
    half = n_tokens // 2
    xr = x.reshape(2, half, d)
    n_steps = half // BLOCK_T

    pa, ia, pb, ib, loss = pl.pallas_call(
        functools.partial(_router_kernel, n_steps=n_steps, n_tokens=n_tokens),
        grid=(n_steps,),
        in_specs=[
            pl.BlockSpec((1, BLOCK_T, d), lambda i: (0, i, 0)),
            pl.BlockSpec((1, BLOCK_T, d), lambda i: (1, i, 0)),
            pl.BlockSpec((N_EXPERTS, d), lambda i: (0, 0)),
        ],
        out_specs=[
            pl.BlockSpec((2, BLOCK_T), lambda i: (0, i)),
            pl.BlockSpec((2, BLOCK_T), lambda i: (0, i)),
            pl.BlockSpec((2, BLOCK_T), lambda i: (0, i)),
            pl.BlockSpec((2, BLOCK_T), lambda i: (0, i)),
            pl.BlockSpec(memory_space=pltpu.SMEM),
        ],
        out_shape=[
            jax.ShapeDtypeStruct((2, half), jnp.float32),
            jax.ShapeDtypeStruct((2, half), jnp.int32),
            jax.ShapeDtypeStruct((2, half), jnp.float32),
            jax.ShapeDtypeStruct((2, half), jnp.int32),
            jax.ShapeDtypeStruct((1, 1), jnp.float32),
        ],
        scratch_shapes=[pltpu.VMEM((N_EXPERTS, 1), jnp.float32)],
    )(xr, xr, W)

    probs = jnp.concatenate([pa, pb], axis=1)
    idx = jnp.concatenate([ia, ib], axis=1)
    return (probs.T.reshape(b, s, 2), idx.T.reshape(b, s, 2), loss[0, 0])
